# Initial kernel scaffold; baseline (speedup 1.0000x reference)
import jax
import jax.numpy as jnp
from jax.experimental import pallas as pl


def kernel(onehot, edge_index, W, b):
    raise NotImplementedError("write your pallas kernel here")


# speedup vs baseline: 20.8989x; 20.8989x over previous
import functools

import jax
import jax.numpy as jnp
from jax import lax
from jax.experimental import pallas as pl
from jax.experimental.pallas import tpu as pltpu
from jax.experimental.pallas import tpu_sc as plsc

NC = 2
NS = 16
K = 128


def _deg_hist_sc(dst, zeros1, ones, n_pad):
    e = dst.shape[0]
    nchunks = e // K
    iters = pl.cdiv(nchunks, NC * NS)
    rt = n_pad // NS
    mesh = plsc.VectorSubcoreMesh(core_axis_name="c", subcore_axis_name="s")

    @functools.partial(
        pl.kernel, mesh=mesh,
        out_type=jax.ShapeDtypeStruct((NC * n_pad,), jnp.float32),
        scratch_types=[
            pltpu.VMEM((K,), jnp.int32),
            pltpu.VMEM((K,), jnp.float32),
            pltpu.VMEM((rt,), jnp.float32),
            pltpu.VMEM_SHARED((n_pad,), jnp.float32),
        ],
    )
    def k(dst_hbm, z_hbm, ones_hbm, out_hbm, idx_v, ones_v, buf_v, acc_sh):
        c = lax.axis_index("c")
        s = lax.axis_index("s")
        w = s * NC + c
        pltpu.sync_copy(z_hbm, buf_v)
        pltpu.sync_copy(buf_v, acc_sh.at[pl.ds(s * rt, rt)])
        pltpu.sync_copy(ones_hbm, ones_v)
        plsc.subcore_barrier()

        def body(i, carry):
            cid = w + i * (NC * NS)

            @pl.when(cid < nchunks)
            def _():
                base = cid * K
                pltpu.sync_copy(dst_hbm.at[pl.ds(base, K)], idx_v)
                pltpu.sync_copy(ones_v, acc_sh.at[idx_v], add=True)

            return carry

        lax.fori_loop(0, iters, body, 0)
        plsc.subcore_barrier()
        pltpu.sync_copy(acc_sh.at[pl.ds(s * rt, rt)], buf_v)
        pltpu.sync_copy(buf_v, out_hbm.at[pl.ds(c * n_pad + s * rt, rt)])

    return k(dst, zeros1, ones)


def _segsum_sc(src, dst, y, zeros2, n_pad, d):
    e = src.shape[0]
    nchunks = e // K
    iters = pl.cdiv(nchunks, NC * NS)
    rt = n_pad // NS
    mesh = plsc.VectorSubcoreMesh(core_axis_name="c", subcore_axis_name="s")

    @functools.partial(
        pl.kernel, mesh=mesh,
        out_type=jax.ShapeDtypeStruct((NC * n_pad, d), jnp.float32),
        scratch_types=[
            pltpu.VMEM((K,), jnp.int32),
            pltpu.VMEM((K,), jnp.int32),
            pltpu.VMEM((K, d), jnp.float32),
            pltpu.VMEM_SHARED((n_pad, d), jnp.float32),
            pltpu.SemaphoreType.DMA,
        ],
    )
    def k(src_hbm, dst_hbm, y_hbm, z_hbm, out_hbm, srcv, dstv, rows_v, acc_sh, sem):
        c = lax.axis_index("c")
        s = lax.axis_index("s")
        w = s * NC + c
        pltpu.sync_copy(z_hbm, rows_v)
        for j in range(rt // K):
            pltpu.sync_copy(rows_v, acc_sh.at[pl.ds(s * rt + j * K, K)])
        plsc.subcore_barrier()

        def body(i, carry):
            cid = w + i * (NC * NS)

            @pl.when(cid < nchunks)
            def _():
                base = cid * K
                pltpu.sync_copy(src_hbm.at[pl.ds(base, K)], srcv)
                pltpu.sync_copy(dst_hbm.at[pl.ds(base, K)], dstv)
                pltpu.async_copy(y_hbm.at[srcv], rows_v, sem).wait()
                pltpu.sync_copy(rows_v, acc_sh.at[dstv], add=True)

            return carry

        lax.fori_loop(0, iters, body, 0)
        plsc.subcore_barrier()
        for j in range(rt // K):
            pltpu.sync_copy(acc_sh.at[pl.ds(s * rt + j * K, K)], rows_v)
            pltpu.sync_copy(rows_v, out_hbm.at[pl.ds(c * n_pad + s * rt + j * K, K)])

    return k(src, dst, y, zeros2)


def _y_tc(onehot, w_mat, p0, p1, n, d, rows):
    def body(x_ref, w_ref, p0_ref, p1_ref, y_ref):
        xw = jnp.dot(x_ref[...], w_ref[...], preferred_element_type=jnp.float32)
        dinv = lax.rsqrt(p0_ref[...] + p1_ref[...] + 1.0)
        y_ref[...] = xw * dinv

    return pl.pallas_call(
        body,
        grid=(n // rows,),
        in_specs=[
            pl.BlockSpec((rows, d), lambda i: (i, 0)),
            pl.BlockSpec((d, d), lambda i: (0, 0)),
            pl.BlockSpec((rows, 1), lambda i: (i, 0)),
            pl.BlockSpec((rows, 1), lambda i: (i, 0)),
        ],
        out_specs=pl.BlockSpec((rows, d), lambda i: (i, 0)),
        out_shape=jax.ShapeDtypeStruct((n, d), jnp.float32),
    )(onehot, w_mat, p0, p1)


def _finish_tc(a0, a1, y, p0, p1, bias, n, d, rows):
    def body(a0_ref, a1_ref, y_ref, p0_ref, p1_ref, b_ref, o_ref):
        dinv = lax.rsqrt(p0_ref[...] + p1_ref[...] + 1.0)
        z = (a0_ref[...] + a1_ref[...] + y_ref[...]) * dinv + b_ref[...]
        z = jnp.maximum(z, 0.0)
        m = jnp.max(z, axis=1, keepdims=True)
        ez = jnp.exp(z - m)
        o_ref[...] = ez / jnp.sum(ez, axis=1, keepdims=True)

    return pl.pallas_call(
        body,
        grid=(n // rows,),
        in_specs=[
            pl.BlockSpec((rows, d), lambda i: (i, 0)),
            pl.BlockSpec((rows, d), lambda i: (i, 0)),
            pl.BlockSpec((rows, d), lambda i: (i, 0)),
            pl.BlockSpec((rows, 1), lambda i: (i, 0)),
            pl.BlockSpec((rows, 1), lambda i: (i, 0)),
            pl.BlockSpec((1, d), lambda i: (0, 0)),
        ],
        out_specs=pl.BlockSpec((rows, d), lambda i: (i, 0)),
        out_shape=jax.ShapeDtypeStruct((n, d), jnp.float32),
    )(a0, a1, y, p0, p1, bias)


def kernel(onehot, edge_index, W, b):
    n, d = onehot.shape
    rt = (-(-n // NS) + K - 1) // K * K
    n_pad = rt * NS
    src = edge_index[0]
    dst = edge_index[1]

    zeros1 = jnp.zeros((rt,), jnp.float32)
    zeros2 = jnp.zeros((K, d), jnp.float32)
    ones = jnp.ones((K,), jnp.float32)

    degp = _deg_hist_sc(dst, zeros1, ones, n_pad)
    p0 = degp[:n][:, None]
    p1 = degp[n_pad:n_pad + n][:, None]
    y = _y_tc(onehot, W, p0, p1, n, d, rows=1000)
    accp = _segsum_sc(src, dst, y, zeros2, n_pad, d)
    return _finish_tc(accp[:n], accp[n_pad:n_pad + n], y, p0, p1,
                      b.reshape(1, d), n, d, rows=1000)
